# baseline (speedup 1.0000x reference)
import functools

import jax
import jax.numpy as jnp
from jax import lax
from jax.experimental import pallas as pl
from jax.experimental.pallas import tpu as pltpu
from jax.experimental.pallas import tpu_sc as plsc

NN = 10000
EE = 320000
DD = 128
HH = 128
GG = 128
LL = 4
NCLS = 10

SC_CORES = 2
SC_TILES = 16
HALF = HH // 2
CHUNK = 128
NCH = 2560
EPAD = NCH * CHUNK
CMAX = NCH // SC_TILES
CMAXD = NCH // (SC_CORES * SC_TILES)
NN_PAD = 10240
RPT = NN_PAD // SC_TILES
DEGW = 16
NBUF = 4
HCH = CMAX // 2
NTAB = 10008
TROW = 624

_sc_mesh = plsc.VectorSubcoreMesh(core_axis_name="c", subcore_axis_name="s")
_sc_params = pltpu.CompilerParams(use_tc_tiling_on_sc=False,
                                 needs_layout_passes=False)


def _zero_rows(ref, nrows, ncols):
    zer = jnp.zeros((16,), jnp.float32)

    def body(i, _):
        for k in range(ncols // 16):
            ref[i, pl.ds(k * 16, 16)] = zer
        return 0

    lax.fori_loop(0, nrows, body, 0)


def _fill_ones_rows(ref, nrows, ncols):
    one = jnp.ones((16,), jnp.float32)

    def body(i, _):
        for k in range(ncols // 16):
            ref[i, pl.ds(k * 16, 16)] = one
        return 0

    lax.fori_loop(0, nrows, body, 0)


@functools.partial(
    pl.kernel,
    out_type=jax.ShapeDtypeStruct((SC_CORES, 2, NN_PAD, DEGW), jnp.float32),
    mesh=_sc_mesh,
    compiler_params=_sc_params,
    scratch_types=[
        pltpu.VMEM_SHARED((NN_PAD, DEGW), jnp.float32),
        pltpu.VMEM_SHARED((NN_PAD, DEGW), jnp.float32),
        pltpu.VMEM((CMAXD, CHUNK), jnp.int32),
        pltpu.VMEM((CMAXD, CHUNK), jnp.int32),
        pltpu.VMEM((CHUNK, DEGW), jnp.float32),
        pltpu.VMEM((RPT, DEGW), jnp.float32),
    ],
)
def _deg_kernel(srcm, dstm, out, dout_sh, din_sh, src_v, dst_v, ones_v, zer_v):
    c = lax.axis_index("c")
    s = lax.axis_index("s")
    w = c * SC_TILES + s

    _fill_ones_rows(ones_v, CHUNK, DEGW)
    _zero_rows(zer_v, RPT, DEGW)
    pltpu.sync_copy(zer_v, dout_sh.at[pl.ds(s * RPT, RPT), :])
    pltpu.sync_copy(zer_v, din_sh.at[pl.ds(s * RPT, RPT), :])
    plsc.subcore_barrier()

    lo = w * CMAXD
    pltpu.sync_copy(srcm.at[pl.ds(lo, CMAXD), :], src_v)
    pltpu.sync_copy(dstm.at[pl.ds(lo, CMAXD), :], dst_v)

    def body(j, _):
        pltpu.sync_copy(ones_v, dout_sh.at[src_v.at[j]], add=True)
        pltpu.sync_copy(ones_v, din_sh.at[dst_v.at[j]], add=True)
        return 0

    lax.fori_loop(0, CMAXD, body, 0)
    plsc.subcore_barrier()

    pltpu.sync_copy(dout_sh.at[pl.ds(s * RPT, RPT), :],
                    out.at[c, 0, pl.ds(s * RPT, RPT), :])
    pltpu.sync_copy(din_sh.at[pl.ds(s * RPT, RPT), :],
                    out.at[c, 1, pl.ds(s * RPT, RPT), :])


@functools.partial(
    pl.kernel,
    out_type=jax.ShapeDtypeStruct((SC_CORES, NN, HALF), jnp.float32),
    mesh=_sc_mesh,
    compiler_params=_sc_params,
    scratch_types=[
        pltpu.VMEM_SHARED((NTAB, HALF), jnp.float32),
        pltpu.VMEM_SHARED((NTAB, HALF), jnp.bfloat16),
        pltpu.VMEM((HCH, CHUNK), jnp.int32),
        pltpu.VMEM((HCH, CHUNK), jnp.int32),
        pltpu.VMEM((NBUF, CHUNK, HALF), jnp.bfloat16),
        pltpu.VMEM((2, CHUNK, HALF), jnp.float32),
        pltpu.VMEM((CHUNK, HALF), jnp.float32),
        pltpu.SemaphoreType.DMA((NBUF,)),
        pltpu.SemaphoreType.DMA((2,)),
    ],
)
def _mp_kernel(xh, srcm, dstm, aggh, agg_sh, x_sh, src_v, dst_v, bf_v, rows_v,
               zer_v, gsem, ssem):
    c = lax.axis_index("c")
    s = lax.axis_index("s")

    pltpu.sync_copy(xh.at[c, pl.ds(s * TROW, TROW), :],
                    x_sh.at[pl.ds(s * TROW, TROW), :])
    _zero_rows(zer_v, CHUNK, HALF)
    for k in range(4):
        pltpu.sync_copy(zer_v,
                        agg_sh.at[pl.ds(s * TROW + k * CHUNK, CHUNK), :])
    pltpu.sync_copy(zer_v.at[pl.ds(0, TROW - 4 * CHUNK), :],
                    agg_sh.at[pl.ds(s * TROW + 4 * CHUNK, TROW - 4 * CHUNK), :])

    @pl.when(s == SC_TILES - 1)
    def _():
        pltpu.sync_copy(xh.at[c, pl.ds(SC_TILES * TROW, NN - SC_TILES * TROW), :],
                        x_sh.at[pl.ds(SC_TILES * TROW, NN - SC_TILES * TROW), :])
        pltpu.sync_copy(zer_v.at[pl.ds(0, NN - SC_TILES * TROW), :],
                        agg_sh.at[pl.ds(SC_TILES * TROW, NN - SC_TILES * TROW), :])

    plsc.subcore_barrier()

    mask_hi = jnp.full((16,), -65536, jnp.int32)

    def widen(b, f):
        @plsc.parallel_loop(0, CHUNK, unroll=16)
        def row(i):
            for g in range(HALF // 32):
                v = bf_v[b, i, pl.ds(g * 32, 32)]
                w = plsc.bitcast(v, jnp.int32)
                lo_f = plsc.bitcast(w << 16, jnp.float32)
                hi_f = plsc.bitcast(w & mask_hi, jnp.float32)
                rows_v[f, i, pl.ds(g * 32, 16)] = lo_f
                rows_v[f, i, pl.ds(g * 32 + 16, 16)] = hi_f

    def body(q, _):
        for b in range(NBUF):
            jb = q * NBUF + b
            pltpu.async_copy(x_sh.at[src_v.at[jb]], bf_v.at[b],
                             gsem.at[b])
        for b in range(NBUF):
            jb = q * NBUF + b
            f = b & 1
            pltpu.make_async_copy(x_sh.at[src_v.at[jb]], bf_v.at[b],
                                  gsem.at[b]).wait()
            if b < 2:
                @pl.when(q > 0)
                def _(f=f, jb=jb):
                    pltpu.make_async_copy(
                        rows_v.at[f], agg_sh.at[dst_v.at[jb - 2]], ssem.at[f]
                    ).wait()
            else:
                pltpu.make_async_copy(
                    rows_v.at[f], agg_sh.at[dst_v.at[jb - 2]], ssem.at[f]
                ).wait()
            widen(b, f)
            pltpu.async_copy(rows_v.at[f], agg_sh.at[dst_v.at[jb]],
                             ssem.at[f], add=True)
        return 0

    for h in range(2):
        base = s * CMAX + h * HCH
        pltpu.sync_copy(srcm.at[pl.ds(base, HCH), :], src_v)
        pltpu.sync_copy(dstm.at[pl.ds(base, HCH), :], dst_v)
        lax.fori_loop(0, HCH // NBUF, body, 0)
        for f in range(2):
            pltpu.make_async_copy(rows_v.at[f],
                                  agg_sh.at[dst_v.at[HCH - 2 + f]],
                                  ssem.at[f]).wait()
    plsc.subcore_barrier()

    pltpu.sync_copy(agg_sh.at[pl.ds(s * TROW, TROW), :],
                    aggh.at[c, pl.ds(s * TROW, TROW), :])

    @pl.when(s == SC_TILES - 1)
    def _():
        pltpu.sync_copy(
            agg_sh.at[pl.ds(SC_TILES * TROW, NN - SC_TILES * TROW), :],
            aggh.at[c, pl.ds(SC_TILES * TROW, NN - SC_TILES * TROW), :])


def _embed_body(nf, w, b, ns, h_out, x_out):
    h = jnp.dot(nf[...], w[...], preferred_element_type=jnp.float32) + b[...]
    h_out[...] = h
    xs = (h * ns[...]).astype(jnp.bfloat16)
    x_out[0] = xs[:, :HALF]
    x_out[1] = xs[:, HALF:]


def _embed_call(nf, w, b, ns):
    return pl.pallas_call(
        _embed_body,
        out_shape=[
            jax.ShapeDtypeStruct((NN, HH), jnp.float32),
            jax.ShapeDtypeStruct((SC_CORES, NN, HALF), jnp.bfloat16),
        ],
    )(nf, w, b, ns)


def _sigma_perm():
    m = lax.broadcasted_iota(jnp.int32, (1, HH), 1)
    blk = m >> 5
    t = m & 31
    src = 32 * blk + jnp.where(t < 16, 2 * t, 2 * (t - 16) + 1)
    r = lax.broadcasted_iota(jnp.int32, (HH, 1), 0)
    return (r == src).astype(jnp.float32)


def _layer_body(agg, h_in, nd, nns, w, b, gamma, beta, ns, h_out, x_out):
    a = agg[...]
    aggf = jnp.concatenate([a[0], a[1]], axis=1) * nd[...]
    dnp = (((0,), (0,)), ((), ()))
    w_eff = lax.dot_general(_sigma_perm(), w[...], dnp,
                            preferred_element_type=jnp.float32)
    hc = jnp.dot(aggf, w_eff, preferred_element_type=jnp.float32) + b[...]
    hc = hc * nns[...]
    mean = jnp.mean(hc, axis=0, keepdims=True)
    cent = hc - mean
    var = jnp.mean(cent * cent, axis=0, keepdims=True)
    hn = cent * lax.rsqrt(var + 1e-5) * gamma[...] + beta[...]
    h = h_in[...] + jnp.maximum(hn, 0.0)
    h_out[...] = h
    xs = (h * ns[...]).astype(jnp.bfloat16)
    x_out[0] = xs[:, :HALF]
    x_out[1] = xs[:, HALF:]


def _layer_call(agg, h_in, nd, nns, w, b, gamma, beta, ns):
    return pl.pallas_call(
        _layer_body,
        out_shape=[
            jax.ShapeDtypeStruct((NN, HH), jnp.float32),
            jax.ShapeDtypeStruct((SC_CORES, NN, HALF), jnp.bfloat16),
        ],
    )(agg, h_in, nd, nns, w, b, gamma, beta, ns)


def _readout_body(h, gid, w0, b0, w1, b1, w2, b2, out):
    iota = lax.broadcasted_iota(jnp.int32, (1, GG), 1)
    onehot = (gid[...] == iota).astype(jnp.float32)
    dn = (((0,), (0,)), ((), ()))
    hsum = lax.dot_general(onehot, h[...], dn,
                           preferred_element_type=jnp.float32)
    counts = lax.dot_general(onehot, jnp.ones((NN, 1), jnp.float32), dn,
                             preferred_element_type=jnp.float32)
    hg = hsum / jnp.maximum(counts, 1.0)
    y = jnp.maximum(jnp.dot(hg, w0[...], preferred_element_type=jnp.float32)
                    + b0[...], 0.0)
    y = jnp.maximum(jnp.dot(y, w1[...], preferred_element_type=jnp.float32)
                    + b1[...], 0.0)
    out[...] = jnp.dot(y, w2[...], preferred_element_type=jnp.float32) + b2[...]


def _readout_call(h, gid, w0, b0, w1, b1, w2, b2):
    return pl.pallas_call(
        _readout_body,
        out_shape=jax.ShapeDtypeStruct((GG, NCLS), jnp.float32),
    )(h, gid, w0, b0, w1, b1, w2, b2)


def kernel(nodes_feat, nodes_num_norm_sqrt, edges_feat, edges_num_norm_sqrt,
           W_embed, b_embed, Ws, bs, gammas, betas,
           W_r0, b_r0, W_r1, b_r1, W_r2, b_r2,
           edge_index, graph_ids):
    pad = jnp.full((2, EPAD - EE), NN, dtype=jnp.int32)
    ei = jnp.concatenate([edge_index, pad], axis=1)
    srcm = ei[0].reshape(NCH, CHUNK)
    dstm = ei[1].reshape(NCH, CHUNK)

    deg = _deg_kernel(srcm, dstm)
    deg_out = deg[0, 0, :NN, 0] + deg[1, 0, :NN, 0]
    deg_in = deg[0, 1, :NN, 0] + deg[1, 1, :NN, 0]
    norm_src = lax.rsqrt(jnp.maximum(deg_out, 1.0)).reshape(NN, 1)
    norm_dst = lax.rsqrt(jnp.maximum(deg_in, 1.0)).reshape(NN, 1)

    h, x = _embed_call(nodes_feat, W_embed, b_embed.reshape(1, HH), norm_src)
    for i in range(LL):
        agg = _mp_kernel(x, srcm, dstm)
        h, x = _layer_call(agg, h, norm_dst, nodes_num_norm_sqrt,
                           Ws[i], bs[i].reshape(1, HH),
                           gammas[i].reshape(1, HH), betas[i].reshape(1, HH),
                           norm_src)

    return _readout_call(h, graph_ids.reshape(NN, 1),
                         W_r0, b_r0.reshape(1, -1),
                         W_r1, b_r1.reshape(1, -1),
                         W_r2, b_r2.reshape(1, -1))


# speedup vs baseline: 1.0135x; 1.0135x over previous
import functools

import jax
import jax.numpy as jnp
from jax import lax
from jax.experimental import pallas as pl
from jax.experimental.pallas import tpu as pltpu
from jax.experimental.pallas import tpu_sc as plsc

NN = 10000
EE = 320000
DD = 128
HH = 128
GG = 128
LL = 4
NCLS = 10

SC_CORES = 2
SC_TILES = 16
HALF = HH // 2
CHUNK = 128
NCH = 2560
EPAD = NCH * CHUNK
CMAX = NCH // SC_TILES
CMAXD = NCH // (SC_CORES * SC_TILES)
NN_PAD = 10240
RPT = NN_PAD // SC_TILES
DEGW = 16
NBUF = 4
HCH = CMAX // 2
NTAB = 10008
TROW = 624

_sc_mesh = plsc.VectorSubcoreMesh(core_axis_name="c", subcore_axis_name="s")
_sc_params = pltpu.CompilerParams(use_tc_tiling_on_sc=False,
                                 needs_layout_passes=False)


def _zero_rows(ref, nrows, ncols):
    zer = jnp.zeros((16,), jnp.float32)

    def body(i, _):
        for k in range(ncols // 16):
            ref[i, pl.ds(k * 16, 16)] = zer
        return 0

    lax.fori_loop(0, nrows, body, 0)


def _fill_ones_rows(ref, nrows, ncols):
    one = jnp.ones((16,), jnp.float32)

    def body(i, _):
        for k in range(ncols // 16):
            ref[i, pl.ds(k * 16, 16)] = one
        return 0

    lax.fori_loop(0, nrows, body, 0)


@functools.partial(
    pl.kernel,
    out_type=jax.ShapeDtypeStruct((SC_CORES, 2, NN_PAD, DEGW), jnp.float32),
    mesh=_sc_mesh,
    compiler_params=_sc_params,
    scratch_types=[
        pltpu.VMEM_SHARED((NN_PAD, DEGW), jnp.float32),
        pltpu.VMEM_SHARED((NN_PAD, DEGW), jnp.float32),
        pltpu.VMEM((CMAXD, CHUNK), jnp.int32),
        pltpu.VMEM((CMAXD, CHUNK), jnp.int32),
        pltpu.VMEM((CHUNK, DEGW), jnp.float32),
        pltpu.VMEM((RPT, DEGW), jnp.float32),
    ],
)
def _deg_kernel(srcm, dstm, out, dout_sh, din_sh, src_v, dst_v, ones_v, zer_v):
    c = lax.axis_index("c")
    s = lax.axis_index("s")
    w = c * SC_TILES + s

    _fill_ones_rows(ones_v, CHUNK, DEGW)
    _zero_rows(zer_v, RPT, DEGW)
    pltpu.sync_copy(zer_v, dout_sh.at[pl.ds(s * RPT, RPT), :])
    pltpu.sync_copy(zer_v, din_sh.at[pl.ds(s * RPT, RPT), :])
    plsc.subcore_barrier()

    lo = w * CMAXD
    pltpu.sync_copy(srcm.at[pl.ds(lo, CMAXD), :], src_v)
    pltpu.sync_copy(dstm.at[pl.ds(lo, CMAXD), :], dst_v)

    def body(j, _):
        pltpu.sync_copy(ones_v, dout_sh.at[src_v.at[j]], add=True)
        pltpu.sync_copy(ones_v, din_sh.at[dst_v.at[j]], add=True)
        return 0

    lax.fori_loop(0, CMAXD, body, 0)
    plsc.subcore_barrier()

    pltpu.sync_copy(dout_sh.at[pl.ds(s * RPT, RPT), :],
                    out.at[c, 0, pl.ds(s * RPT, RPT), :])
    pltpu.sync_copy(din_sh.at[pl.ds(s * RPT, RPT), :],
                    out.at[c, 1, pl.ds(s * RPT, RPT), :])


@functools.partial(
    pl.kernel,
    out_type=jax.ShapeDtypeStruct((SC_CORES, NN, HALF), jnp.float32),
    mesh=_sc_mesh,
    compiler_params=_sc_params,
    scratch_types=[
        pltpu.VMEM_SHARED((NTAB, HALF), jnp.float32),
        pltpu.VMEM_SHARED((NTAB, HALF), jnp.bfloat16),
        pltpu.VMEM((HCH, CHUNK), jnp.int32),
        pltpu.VMEM((HCH, CHUNK), jnp.int32),
        pltpu.VMEM((NBUF, CHUNK, HALF), jnp.bfloat16),
        pltpu.VMEM((2, CHUNK, HALF), jnp.float32),
        pltpu.VMEM((CHUNK, HALF), jnp.float32),
        pltpu.SemaphoreType.DMA((NBUF,)),
        pltpu.SemaphoreType.DMA((2,)),
    ],
)
def _mp_kernel(xh, srcm, dstm, aggh, agg_sh, x_sh, src_v, dst_v, bf_v, rows_v,
               zer_v, gsem, ssem):
    c = lax.axis_index("c")
    s = lax.axis_index("s")

    pltpu.sync_copy(xh.at[c, pl.ds(s * TROW, TROW), :],
                    x_sh.at[pl.ds(s * TROW, TROW), :])
    _zero_rows(zer_v, CHUNK, HALF)
    for k in range(4):
        pltpu.sync_copy(zer_v,
                        agg_sh.at[pl.ds(s * TROW + k * CHUNK, CHUNK), :])
    pltpu.sync_copy(zer_v.at[pl.ds(0, TROW - 4 * CHUNK), :],
                    agg_sh.at[pl.ds(s * TROW + 4 * CHUNK, TROW - 4 * CHUNK), :])

    @pl.when(s == SC_TILES - 1)
    def _():
        pltpu.sync_copy(xh.at[c, pl.ds(SC_TILES * TROW, NN - SC_TILES * TROW), :],
                        x_sh.at[pl.ds(SC_TILES * TROW, NN - SC_TILES * TROW), :])
        pltpu.sync_copy(zer_v.at[pl.ds(0, NN - SC_TILES * TROW), :],
                        agg_sh.at[pl.ds(SC_TILES * TROW, NN - SC_TILES * TROW), :])

    plsc.subcore_barrier()

    mask_hi = jnp.full((16,), -65536, jnp.int32)

    def widen(b, f):
        @plsc.parallel_loop(0, CHUNK, unroll=8)
        def row(i):
            for g in range(HALF // 32):
                v = bf_v[b, i, pl.ds(g * 32, 32)]
                w = plsc.bitcast(v, jnp.int32)
                lo_f = plsc.bitcast(w << 16, jnp.float32)
                hi_f = plsc.bitcast(w & mask_hi, jnp.float32)
                rows_v[f, i, pl.ds(g * 32, 16)] = lo_f
                rows_v[f, i, pl.ds(g * 32 + 16, 16)] = hi_f

    def body(q, _):
        for b in range(NBUF):
            jb = q * NBUF + b
            pltpu.async_copy(x_sh.at[src_v.at[jb]], bf_v.at[b],
                             gsem.at[b])
        for b in range(NBUF):
            jb = q * NBUF + b
            f = b & 1
            pltpu.make_async_copy(x_sh.at[src_v.at[jb]], bf_v.at[b],
                                  gsem.at[b]).wait()
            if b < 2:
                @pl.when(q > 0)
                def _(f=f, jb=jb):
                    pltpu.make_async_copy(
                        rows_v.at[f], agg_sh.at[dst_v.at[jb - 2]], ssem.at[f]
                    ).wait()
            else:
                pltpu.make_async_copy(
                    rows_v.at[f], agg_sh.at[dst_v.at[jb - 2]], ssem.at[f]
                ).wait()
            widen(b, f)
            pltpu.async_copy(rows_v.at[f], agg_sh.at[dst_v.at[jb]],
                             ssem.at[f], add=True)
        return 0

    for h in range(2):
        base = s * CMAX + h * HCH
        pltpu.sync_copy(srcm.at[pl.ds(base, HCH), :], src_v)
        pltpu.sync_copy(dstm.at[pl.ds(base, HCH), :], dst_v)
        lax.fori_loop(0, HCH // NBUF, body, 0)
        for f in range(2):
            pltpu.make_async_copy(rows_v.at[f],
                                  agg_sh.at[dst_v.at[HCH - 2 + f]],
                                  ssem.at[f]).wait()
    plsc.subcore_barrier()

    pltpu.sync_copy(agg_sh.at[pl.ds(s * TROW, TROW), :],
                    aggh.at[c, pl.ds(s * TROW, TROW), :])

    @pl.when(s == SC_TILES - 1)
    def _():
        pltpu.sync_copy(
            agg_sh.at[pl.ds(SC_TILES * TROW, NN - SC_TILES * TROW), :],
            aggh.at[c, pl.ds(SC_TILES * TROW, NN - SC_TILES * TROW), :])


def _embed_body(nf, w, b, ns, h_out, x_out):
    h = jnp.dot(nf[...], w[...], preferred_element_type=jnp.float32) + b[...]
    h_out[...] = h
    xs = (h * ns[...]).astype(jnp.bfloat16)
    x_out[0] = xs[:, :HALF]
    x_out[1] = xs[:, HALF:]


def _embed_call(nf, w, b, ns):
    return pl.pallas_call(
        _embed_body,
        out_shape=[
            jax.ShapeDtypeStruct((NN, HH), jnp.float32),
            jax.ShapeDtypeStruct((SC_CORES, NN, HALF), jnp.bfloat16),
        ],
    )(nf, w, b, ns)


def _sigma_perm():
    m = lax.broadcasted_iota(jnp.int32, (1, HH), 1)
    blk = m >> 5
    t = m & 31
    src = 32 * blk + jnp.where(t < 16, 2 * t, 2 * (t - 16) + 1)
    r = lax.broadcasted_iota(jnp.int32, (HH, 1), 0)
    return (r == src).astype(jnp.float32)


def _layer_body(agg, h_in, nd, nns, w, b, gamma, beta, ns, h_out, x_out):
    a = agg[...]
    aggf = jnp.concatenate([a[0], a[1]], axis=1) * nd[...]
    dnp = (((0,), (0,)), ((), ()))
    w_eff = lax.dot_general(_sigma_perm(), w[...], dnp,
                            preferred_element_type=jnp.float32)
    hc = jnp.dot(aggf, w_eff, preferred_element_type=jnp.float32) + b[...]
    hc = hc * nns[...]
    mean = jnp.mean(hc, axis=0, keepdims=True)
    cent = hc - mean
    var = jnp.mean(cent * cent, axis=0, keepdims=True)
    hn = cent * lax.rsqrt(var + 1e-5) * gamma[...] + beta[...]
    h = h_in[...] + jnp.maximum(hn, 0.0)
    h_out[...] = h
    xs = (h * ns[...]).astype(jnp.bfloat16)
    x_out[0] = xs[:, :HALF]
    x_out[1] = xs[:, HALF:]


def _layer_call(agg, h_in, nd, nns, w, b, gamma, beta, ns):
    return pl.pallas_call(
        _layer_body,
        out_shape=[
            jax.ShapeDtypeStruct((NN, HH), jnp.float32),
            jax.ShapeDtypeStruct((SC_CORES, NN, HALF), jnp.bfloat16),
        ],
    )(agg, h_in, nd, nns, w, b, gamma, beta, ns)


def _readout_body(h, gid, w0, b0, w1, b1, w2, b2, out):
    iota = lax.broadcasted_iota(jnp.int32, (1, GG), 1)
    onehot = (gid[...] == iota).astype(jnp.float32)
    dn = (((0,), (0,)), ((), ()))
    hsum = lax.dot_general(onehot, h[...], dn,
                           preferred_element_type=jnp.float32)
    counts = lax.dot_general(onehot, jnp.ones((NN, 1), jnp.float32), dn,
                             preferred_element_type=jnp.float32)
    hg = hsum / jnp.maximum(counts, 1.0)
    y = jnp.maximum(jnp.dot(hg, w0[...], preferred_element_type=jnp.float32)
                    + b0[...], 0.0)
    y = jnp.maximum(jnp.dot(y, w1[...], preferred_element_type=jnp.float32)
                    + b1[...], 0.0)
    out[...] = jnp.dot(y, w2[...], preferred_element_type=jnp.float32) + b2[...]


def _readout_call(h, gid, w0, b0, w1, b1, w2, b2):
    return pl.pallas_call(
        _readout_body,
        out_shape=jax.ShapeDtypeStruct((GG, NCLS), jnp.float32),
    )(h, gid, w0, b0, w1, b1, w2, b2)


def kernel(nodes_feat, nodes_num_norm_sqrt, edges_feat, edges_num_norm_sqrt,
           W_embed, b_embed, Ws, bs, gammas, betas,
           W_r0, b_r0, W_r1, b_r1, W_r2, b_r2,
           edge_index, graph_ids):
    pad = jnp.full((2, EPAD - EE), NN, dtype=jnp.int32)
    ei = jnp.concatenate([edge_index, pad], axis=1)
    srcm = ei[0].reshape(NCH, CHUNK)
    dstm = ei[1].reshape(NCH, CHUNK)

    deg = _deg_kernel(srcm, dstm)
    deg_out = deg[0, 0, :NN, 0] + deg[1, 0, :NN, 0]
    deg_in = deg[0, 1, :NN, 0] + deg[1, 1, :NN, 0]
    norm_src = lax.rsqrt(jnp.maximum(deg_out, 1.0)).reshape(NN, 1)
    norm_dst = lax.rsqrt(jnp.maximum(deg_in, 1.0)).reshape(NN, 1)

    h, x = _embed_call(nodes_feat, W_embed, b_embed.reshape(1, HH), norm_src)
    for i in range(LL):
        agg = _mp_kernel(x, srcm, dstm)
        h, x = _layer_call(agg, h, norm_dst, nodes_num_norm_sqrt,
                           Ws[i], bs[i].reshape(1, HH),
                           gammas[i].reshape(1, HH), betas[i].reshape(1, HH),
                           norm_src)

    return _readout_call(h, graph_ids.reshape(NN, 1),
                         W_r0, b_r0.reshape(1, -1),
                         W_r1, b_r1.reshape(1, -1),
                         W_r2, b_r2.reshape(1, -1))


# speedup vs baseline: 1.0264x; 1.0127x over previous
import functools

import jax
import jax.numpy as jnp
from jax import lax
from jax.experimental import pallas as pl
from jax.experimental.pallas import tpu as pltpu
from jax.experimental.pallas import tpu_sc as plsc

NN = 10000
EE = 320000
DD = 128
HH = 128
GG = 128
LL = 4
NCLS = 10

SC_CORES = 2
SC_TILES = 16
HALF = HH // 2
CHUNK = 128
NCH = 2560
EPAD = NCH * CHUNK
CMAX = NCH // SC_TILES
CMAXD = NCH // (SC_CORES * SC_TILES)
NN_PAD = 10240
RPT = NN_PAD // SC_TILES
DEGW = 16
NBUF = 4
HCH = CMAX // 2
NTAB = 10008
TROW = 624

_sc_mesh = plsc.VectorSubcoreMesh(core_axis_name="c", subcore_axis_name="s")
_sc_params = pltpu.CompilerParams(use_tc_tiling_on_sc=False,
                                 needs_layout_passes=False)


def _zero_rows(ref, nrows, ncols):
    zer = jnp.zeros((16,), jnp.float32)

    def body(i, _):
        for k in range(ncols // 16):
            ref[i, pl.ds(k * 16, 16)] = zer
        return 0

    lax.fori_loop(0, nrows, body, 0)


def _fill_ones_rows(ref, nrows, ncols):
    one = jnp.ones((16,), jnp.float32)

    def body(i, _):
        for k in range(ncols // 16):
            ref[i, pl.ds(k * 16, 16)] = one
        return 0

    lax.fori_loop(0, nrows, body, 0)


@functools.partial(
    pl.kernel,
    out_type=jax.ShapeDtypeStruct((SC_CORES, 2, NN_PAD, DEGW), jnp.float32),
    mesh=_sc_mesh,
    compiler_params=_sc_params,
    scratch_types=[
        pltpu.VMEM_SHARED((NN_PAD, DEGW), jnp.float32),
        pltpu.VMEM_SHARED((NN_PAD, DEGW), jnp.float32),
        pltpu.VMEM((CMAXD, CHUNK), jnp.int32),
        pltpu.VMEM((CMAXD, CHUNK), jnp.int32),
        pltpu.VMEM((CHUNK, DEGW), jnp.float32),
        pltpu.VMEM((RPT, DEGW), jnp.float32),
    ],
)
def _deg_kernel(srcm, dstm, out, dout_sh, din_sh, src_v, dst_v, ones_v, zer_v):
    c = lax.axis_index("c")
    s = lax.axis_index("s")
    w = c * SC_TILES + s

    _fill_ones_rows(ones_v, CHUNK, DEGW)
    _zero_rows(zer_v, RPT, DEGW)
    pltpu.sync_copy(zer_v, dout_sh.at[pl.ds(s * RPT, RPT), :])
    pltpu.sync_copy(zer_v, din_sh.at[pl.ds(s * RPT, RPT), :])
    plsc.subcore_barrier()

    lo = w * CMAXD
    pltpu.sync_copy(srcm.at[pl.ds(lo, CMAXD), :], src_v)
    pltpu.sync_copy(dstm.at[pl.ds(lo, CMAXD), :], dst_v)

    def body(j, _):
        pltpu.sync_copy(ones_v, dout_sh.at[src_v.at[j]], add=True)
        pltpu.sync_copy(ones_v, din_sh.at[dst_v.at[j]], add=True)
        return 0

    lax.fori_loop(0, CMAXD, body, 0)
    plsc.subcore_barrier()

    pltpu.sync_copy(dout_sh.at[pl.ds(s * RPT, RPT), :],
                    out.at[c, 0, pl.ds(s * RPT, RPT), :])
    pltpu.sync_copy(din_sh.at[pl.ds(s * RPT, RPT), :],
                    out.at[c, 1, pl.ds(s * RPT, RPT), :])


@functools.partial(
    pl.kernel,
    out_type=jax.ShapeDtypeStruct((SC_CORES, NN, HALF), jnp.float32),
    mesh=_sc_mesh,
    compiler_params=_sc_params,
    scratch_types=[
        pltpu.VMEM_SHARED((NTAB, HALF), jnp.float32),
        pltpu.VMEM_SHARED((NTAB, HALF), jnp.bfloat16),
        pltpu.VMEM((HCH, CHUNK), jnp.int32),
        pltpu.VMEM((HCH, CHUNK), jnp.int32),
        pltpu.VMEM((NBUF, CHUNK, HALF), jnp.bfloat16),
        pltpu.VMEM((2, CHUNK, HALF), jnp.float32),
        pltpu.VMEM((CHUNK, HALF), jnp.float32),
        pltpu.SemaphoreType.DMA((NBUF,)),
        pltpu.SemaphoreType.DMA((2,)),
    ],
)
def _mp_kernel(xh, srcm, dstm, aggh, agg_sh, x_sh, src_v, dst_v, bf_v, rows_v,
               zer_v, gsem, ssem):
    c = lax.axis_index("c")
    s = lax.axis_index("s")

    pltpu.sync_copy(xh.at[c, pl.ds(s * TROW, TROW), :],
                    x_sh.at[pl.ds(s * TROW, TROW), :])
    _zero_rows(zer_v, CHUNK, HALF)
    for k in range(4):
        pltpu.sync_copy(zer_v,
                        agg_sh.at[pl.ds(s * TROW + k * CHUNK, CHUNK), :])
    pltpu.sync_copy(zer_v.at[pl.ds(0, TROW - 4 * CHUNK), :],
                    agg_sh.at[pl.ds(s * TROW + 4 * CHUNK, TROW - 4 * CHUNK), :])

    @pl.when(s == SC_TILES - 1)
    def _():
        pltpu.sync_copy(xh.at[c, pl.ds(SC_TILES * TROW, NN - SC_TILES * TROW), :],
                        x_sh.at[pl.ds(SC_TILES * TROW, NN - SC_TILES * TROW), :])
        pltpu.sync_copy(zer_v.at[pl.ds(0, NN - SC_TILES * TROW), :],
                        agg_sh.at[pl.ds(SC_TILES * TROW, NN - SC_TILES * TROW), :])

    plsc.subcore_barrier()

    mask_hi = jnp.full((16,), -65536, jnp.int32)

    def widen(b, f):
        @plsc.parallel_loop(0, CHUNK, unroll=8)
        def row(i):
            for g in range(HALF // 32):
                v = bf_v[b, i, pl.ds(g * 32, 32)]
                w = plsc.bitcast(v, jnp.int32)
                lo_f = plsc.bitcast(w << 16, jnp.float32)
                hi_f = plsc.bitcast(w & mask_hi, jnp.float32)
                rows_v[f, i, pl.ds(g * 32, 16)] = lo_f
                rows_v[f, i, pl.ds(g * 32 + 16, 16)] = hi_f

    def body(q, _):
        for b in range(NBUF):
            jb = q * NBUF + b
            pltpu.async_copy(x_sh.at[src_v.at[jb]], bf_v.at[b],
                             gsem.at[b])
        for b in range(NBUF):
            jb = q * NBUF + b
            f = b & 1
            pltpu.make_async_copy(x_sh.at[src_v.at[jb]], bf_v.at[b],
                                  gsem.at[b]).wait()
            if b < 2:
                @pl.when(q > 0)
                def _(f=f, jb=jb):
                    pltpu.make_async_copy(
                        rows_v.at[f], agg_sh.at[dst_v.at[jb - 2]], ssem.at[f]
                    ).wait()
            else:
                pltpu.make_async_copy(
                    rows_v.at[f], agg_sh.at[dst_v.at[jb - 2]], ssem.at[f]
                ).wait()
            widen(b, f)
            pltpu.async_copy(rows_v.at[f], agg_sh.at[dst_v.at[jb]],
                             ssem.at[f], add=True)
        return 0

    for h in range(2):
        base = s * CMAX + h * HCH
        pltpu.sync_copy(srcm.at[pl.ds(base, HCH), :], src_v)
        pltpu.sync_copy(dstm.at[pl.ds(base, HCH), :], dst_v)
        lax.fori_loop(0, HCH // NBUF, body, 0)
        for f in range(2):
            pltpu.make_async_copy(rows_v.at[f],
                                  agg_sh.at[dst_v.at[HCH - 2 + f]],
                                  ssem.at[f]).wait()
    plsc.subcore_barrier()

    pltpu.sync_copy(agg_sh.at[pl.ds(s * TROW, TROW), :],
                    aggh.at[c, pl.ds(s * TROW, TROW), :])

    @pl.when(s == SC_TILES - 1)
    def _():
        pltpu.sync_copy(
            agg_sh.at[pl.ds(SC_TILES * TROW, NN - SC_TILES * TROW), :],
            aggh.at[c, pl.ds(SC_TILES * TROW, NN - SC_TILES * TROW), :])


def _embed_body(nf, w, b, ns, h_out, x_out):
    h = jnp.dot(nf[...], w[...], preferred_element_type=jnp.float32) + b[...]
    h_out[...] = h
    xs = (h * ns[...]).astype(jnp.bfloat16)
    x_out[0] = xs[:, :HALF]
    x_out[1] = xs[:, HALF:]


def _embed_call(nf, w, b, ns):
    return pl.pallas_call(
        _embed_body,
        out_shape=[
            jax.ShapeDtypeStruct((NN, HH), jnp.float32),
            jax.ShapeDtypeStruct((SC_CORES, NN, HALF), jnp.bfloat16),
        ],
    )(nf, w, b, ns)


def _sigma_perm():
    m = lax.broadcasted_iota(jnp.int32, (1, HH), 1)
    blk = m >> 5
    t = m & 31
    src = 32 * blk + jnp.where(t < 16, 2 * t, 2 * (t - 16) + 1)
    r = lax.broadcasted_iota(jnp.int32, (HH, 1), 0)
    return (r == src).astype(jnp.float32)


def _layer_core(agg, h_in, nd, nns, w, b, gamma, beta):
    a = agg[...]
    aggf = jnp.concatenate([a[0], a[1]], axis=1) * nd[...]
    dnp = (((0,), (0,)), ((), ()))
    w_eff = lax.dot_general(_sigma_perm(), w[...], dnp,
                            preferred_element_type=jnp.float32)
    hc = jnp.dot(aggf, w_eff, preferred_element_type=jnp.float32) + b[...]
    hc = hc * nns[...]
    mean = jnp.mean(hc, axis=0, keepdims=True)
    cent = hc - mean
    var = jnp.mean(cent * cent, axis=0, keepdims=True)
    hn = cent * lax.rsqrt(var + 1e-5) * gamma[...] + beta[...]
    return h_in[...] + jnp.maximum(hn, 0.0)


def _layer_body(agg, h_in, nd, nns, w, b, gamma, beta, ns, h_out, x_out):
    h = _layer_core(agg, h_in, nd, nns, w, b, gamma, beta)
    h_out[...] = h
    xs = (h * ns[...]).astype(jnp.bfloat16)
    x_out[0] = xs[:, :HALF]
    x_out[1] = xs[:, HALF:]


def _layer_call(agg, h_in, nd, nns, w, b, gamma, beta, ns):
    return pl.pallas_call(
        _layer_body,
        out_shape=[
            jax.ShapeDtypeStruct((NN, HH), jnp.float32),
            jax.ShapeDtypeStruct((SC_CORES, NN, HALF), jnp.bfloat16),
        ],
    )(agg, h_in, nd, nns, w, b, gamma, beta, ns)


def _last_body(agg, h_in, nd, nns, w, b, gamma, beta,
               gid, w0, b0, w1, b1, w2, b2, out):
    h = _layer_core(agg, h_in, nd, nns, w, b, gamma, beta)
    iota = lax.broadcasted_iota(jnp.int32, (1, GG), 1)
    onehot = (gid[...] == iota).astype(jnp.float32)
    dn = (((0,), (0,)), ((), ()))
    hsum = lax.dot_general(onehot, h, dn,
                           preferred_element_type=jnp.float32)
    counts = lax.dot_general(onehot, jnp.ones((NN, 1), jnp.float32), dn,
                             preferred_element_type=jnp.float32)
    hg = hsum / jnp.maximum(counts, 1.0)
    y = jnp.maximum(jnp.dot(hg, w0[...], preferred_element_type=jnp.float32)
                    + b0[...], 0.0)
    y = jnp.maximum(jnp.dot(y, w1[...], preferred_element_type=jnp.float32)
                    + b1[...], 0.0)
    out[...] = jnp.dot(y, w2[...], preferred_element_type=jnp.float32) + b2[...]


def _last_call(agg, h_in, nd, nns, w, b, gamma, beta,
               gid, w0, b0, w1, b1, w2, b2):
    return pl.pallas_call(
        _last_body,
        out_shape=jax.ShapeDtypeStruct((GG, NCLS), jnp.float32),
    )(agg, h_in, nd, nns, w, b, gamma, beta, gid, w0, b0, w1, b1, w2, b2)


def _readout_body(h, gid, w0, b0, w1, b1, w2, b2, out):
    iota = lax.broadcasted_iota(jnp.int32, (1, GG), 1)
    onehot = (gid[...] == iota).astype(jnp.float32)
    dn = (((0,), (0,)), ((), ()))
    hsum = lax.dot_general(onehot, h[...], dn,
                           preferred_element_type=jnp.float32)
    counts = lax.dot_general(onehot, jnp.ones((NN, 1), jnp.float32), dn,
                             preferred_element_type=jnp.float32)
    hg = hsum / jnp.maximum(counts, 1.0)
    y = jnp.maximum(jnp.dot(hg, w0[...], preferred_element_type=jnp.float32)
                    + b0[...], 0.0)
    y = jnp.maximum(jnp.dot(y, w1[...], preferred_element_type=jnp.float32)
                    + b1[...], 0.0)
    out[...] = jnp.dot(y, w2[...], preferred_element_type=jnp.float32) + b2[...]


def _readout_call(h, gid, w0, b0, w1, b1, w2, b2):
    return pl.pallas_call(
        _readout_body,
        out_shape=jax.ShapeDtypeStruct((GG, NCLS), jnp.float32),
    )(h, gid, w0, b0, w1, b1, w2, b2)


def kernel(nodes_feat, nodes_num_norm_sqrt, edges_feat, edges_num_norm_sqrt,
           W_embed, b_embed, Ws, bs, gammas, betas,
           W_r0, b_r0, W_r1, b_r1, W_r2, b_r2,
           edge_index, graph_ids):
    pad = jnp.full((2, EPAD - EE), NN, dtype=jnp.int32)
    ei = jnp.concatenate([edge_index, pad], axis=1)
    srcm = ei[0].reshape(NCH, CHUNK)
    dstm = ei[1].reshape(NCH, CHUNK)

    deg = _deg_kernel(srcm, dstm)
    deg_out = deg[0, 0, :NN, 0] + deg[1, 0, :NN, 0]
    deg_in = deg[0, 1, :NN, 0] + deg[1, 1, :NN, 0]
    norm_src = lax.rsqrt(jnp.maximum(deg_out, 1.0)).reshape(NN, 1)
    norm_dst = lax.rsqrt(jnp.maximum(deg_in, 1.0)).reshape(NN, 1)

    h, x = _embed_call(nodes_feat, W_embed, b_embed.reshape(1, HH), norm_src)
    for i in range(LL - 1):
        agg = _mp_kernel(x, srcm, dstm)
        h, x = _layer_call(agg, h, norm_dst, nodes_num_norm_sqrt,
                           Ws[i], bs[i].reshape(1, HH),
                           gammas[i].reshape(1, HH), betas[i].reshape(1, HH),
                           norm_src)

    agg = _mp_kernel(x, srcm, dstm)
    i = LL - 1
    return _last_call(agg, h, norm_dst, nodes_num_norm_sqrt,
                      Ws[i], bs[i].reshape(1, HH),
                      gammas[i].reshape(1, HH), betas[i].reshape(1, HH),
                      graph_ids.reshape(NN, 1),
                      W_r0, b_r0.reshape(1, -1),
                      W_r1, b_r1.reshape(1, -1),
                      W_r2, b_r2.reshape(1, -1))


# speedup vs baseline: 1.0299x; 1.0034x over previous
import functools

import jax
import jax.numpy as jnp
from jax import lax
from jax.experimental import pallas as pl
from jax.experimental.pallas import tpu as pltpu
from jax.experimental.pallas import tpu_sc as plsc

NN = 10000
EE = 320000
DD = 128
HH = 128
GG = 128
LL = 4
NCLS = 10

SC_CORES = 2
SC_TILES = 16
HALF = HH // 2
CHUNK = 128
NCH = 2560
EPAD = NCH * CHUNK
CMAX = NCH // SC_TILES
CMAXD = NCH // (SC_CORES * SC_TILES)
NN_PAD = 10240
RPT = NN_PAD // SC_TILES
DEGW = 16
NBUF = 4
HCH = CMAX // 2
NTAB = 10008
TROW = 624

_sc_mesh = plsc.VectorSubcoreMesh(core_axis_name="c", subcore_axis_name="s")
_sc_params = pltpu.CompilerParams(use_tc_tiling_on_sc=False,
                                 needs_layout_passes=False)


def _zero_rows(ref, nrows, ncols):
    zer = jnp.zeros((16,), jnp.float32)

    def body(i, _):
        for k in range(ncols // 16):
            ref[i, pl.ds(k * 16, 16)] = zer
        return 0

    lax.fori_loop(0, nrows, body, 0)


def _fill_ones_rows(ref, nrows, ncols):
    one = jnp.ones((16,), jnp.float32)

    def body(i, _):
        for k in range(ncols // 16):
            ref[i, pl.ds(k * 16, 16)] = one
        return 0

    lax.fori_loop(0, nrows, body, 0)


@functools.partial(
    pl.kernel,
    out_type=jax.ShapeDtypeStruct((SC_CORES, 2, NN_PAD, DEGW), jnp.float32),
    mesh=_sc_mesh,
    compiler_params=_sc_params,
    scratch_types=[
        pltpu.VMEM_SHARED((NN_PAD, DEGW), jnp.float32),
        pltpu.VMEM_SHARED((NN_PAD, DEGW), jnp.float32),
        pltpu.VMEM((CMAXD, CHUNK), jnp.int32),
        pltpu.VMEM((CMAXD, CHUNK), jnp.int32),
        pltpu.VMEM((CHUNK, DEGW), jnp.float32),
        pltpu.VMEM((RPT, DEGW), jnp.float32),
        pltpu.SemaphoreType.DMA,
        pltpu.SemaphoreType.DMA,
    ],
)
def _deg_kernel(srcm, dstm, out, dout_sh, din_sh, src_v, dst_v, ones_v, zer_v,
                osem, isem):
    c = lax.axis_index("c")
    s = lax.axis_index("s")
    w = c * SC_TILES + s

    _fill_ones_rows(ones_v, CHUNK, DEGW)
    _zero_rows(zer_v, RPT, DEGW)
    pltpu.sync_copy(zer_v, dout_sh.at[pl.ds(s * RPT, RPT), :])
    pltpu.sync_copy(zer_v, din_sh.at[pl.ds(s * RPT, RPT), :])
    plsc.subcore_barrier()

    lo = w * CMAXD
    pltpu.sync_copy(srcm.at[pl.ds(lo, CMAXD), :], src_v)
    pltpu.sync_copy(dstm.at[pl.ds(lo, CMAXD), :], dst_v)

    def body(j, _):
        pltpu.async_copy(ones_v, dout_sh.at[src_v.at[j]], osem, add=True)
        pltpu.async_copy(ones_v, din_sh.at[dst_v.at[j]], isem, add=True)
        return 0

    lax.fori_loop(0, CMAXD, body, 0)

    def drain(j, _):
        pltpu.make_async_copy(ones_v, dout_sh.at[src_v.at[j]], osem).wait()
        pltpu.make_async_copy(ones_v, din_sh.at[dst_v.at[j]], isem).wait()
        return 0

    lax.fori_loop(0, CMAXD, drain, 0)
    plsc.subcore_barrier()

    pltpu.sync_copy(dout_sh.at[pl.ds(s * RPT, RPT), :],
                    out.at[c, 0, pl.ds(s * RPT, RPT), :])
    pltpu.sync_copy(din_sh.at[pl.ds(s * RPT, RPT), :],
                    out.at[c, 1, pl.ds(s * RPT, RPT), :])


@functools.partial(
    pl.kernel,
    out_type=jax.ShapeDtypeStruct((SC_CORES, NN, HALF), jnp.float32),
    mesh=_sc_mesh,
    compiler_params=_sc_params,
    scratch_types=[
        pltpu.VMEM_SHARED((NTAB, HALF), jnp.float32),
        pltpu.VMEM_SHARED((NTAB, HALF), jnp.bfloat16),
        pltpu.VMEM((HCH, CHUNK), jnp.int32),
        pltpu.VMEM((HCH, CHUNK), jnp.int32),
        pltpu.VMEM((NBUF, CHUNK, HALF), jnp.bfloat16),
        pltpu.VMEM((2, CHUNK, HALF), jnp.float32),
        pltpu.VMEM((CHUNK, HALF), jnp.float32),
        pltpu.SemaphoreType.DMA((NBUF,)),
        pltpu.SemaphoreType.DMA((2,)),
    ],
)
def _mp_kernel(xh, srcm, dstm, aggh, agg_sh, x_sh, src_v, dst_v, bf_v, rows_v,
               zer_v, gsem, ssem):
    c = lax.axis_index("c")
    s = lax.axis_index("s")

    pltpu.sync_copy(xh.at[c, pl.ds(s * TROW, TROW), :],
                    x_sh.at[pl.ds(s * TROW, TROW), :])
    _zero_rows(zer_v, CHUNK, HALF)
    for k in range(4):
        pltpu.sync_copy(zer_v,
                        agg_sh.at[pl.ds(s * TROW + k * CHUNK, CHUNK), :])
    pltpu.sync_copy(zer_v.at[pl.ds(0, TROW - 4 * CHUNK), :],
                    agg_sh.at[pl.ds(s * TROW + 4 * CHUNK, TROW - 4 * CHUNK), :])

    @pl.when(s == SC_TILES - 1)
    def _():
        pltpu.sync_copy(xh.at[c, pl.ds(SC_TILES * TROW, NN - SC_TILES * TROW), :],
                        x_sh.at[pl.ds(SC_TILES * TROW, NN - SC_TILES * TROW), :])
        pltpu.sync_copy(zer_v.at[pl.ds(0, NN - SC_TILES * TROW), :],
                        agg_sh.at[pl.ds(SC_TILES * TROW, NN - SC_TILES * TROW), :])

    plsc.subcore_barrier()

    mask_hi = jnp.full((16,), -65536, jnp.int32)

    def widen(b, f):
        @plsc.parallel_loop(0, CHUNK, unroll=8)
        def row(i):
            for g in range(HALF // 32):
                v = bf_v[b, i, pl.ds(g * 32, 32)]
                w = plsc.bitcast(v, jnp.int32)
                lo_f = plsc.bitcast(w << 16, jnp.float32)
                hi_f = plsc.bitcast(w & mask_hi, jnp.float32)
                rows_v[f, i, pl.ds(g * 32, 16)] = lo_f
                rows_v[f, i, pl.ds(g * 32 + 16, 16)] = hi_f

    def body(q, _):
        for b in range(NBUF):
            jb = q * NBUF + b
            pltpu.async_copy(x_sh.at[src_v.at[jb]], bf_v.at[b],
                             gsem.at[b])
        for b in range(NBUF):
            jb = q * NBUF + b
            f = b & 1
            pltpu.make_async_copy(x_sh.at[src_v.at[jb]], bf_v.at[b],
                                  gsem.at[b]).wait()
            if b < 2:
                @pl.when(q > 0)
                def _(f=f, jb=jb):
                    pltpu.make_async_copy(
                        rows_v.at[f], agg_sh.at[dst_v.at[jb - 2]], ssem.at[f]
                    ).wait()
            else:
                pltpu.make_async_copy(
                    rows_v.at[f], agg_sh.at[dst_v.at[jb - 2]], ssem.at[f]
                ).wait()
            widen(b, f)
            pltpu.async_copy(rows_v.at[f], agg_sh.at[dst_v.at[jb]],
                             ssem.at[f], add=True)
        return 0

    for h in range(2):
        base = s * CMAX + h * HCH
        pltpu.sync_copy(srcm.at[pl.ds(base, HCH), :], src_v)
        pltpu.sync_copy(dstm.at[pl.ds(base, HCH), :], dst_v)
        lax.fori_loop(0, HCH // NBUF, body, 0)
        for f in range(2):
            pltpu.make_async_copy(rows_v.at[f],
                                  agg_sh.at[dst_v.at[HCH - 2 + f]],
                                  ssem.at[f]).wait()
    plsc.subcore_barrier()

    pltpu.sync_copy(agg_sh.at[pl.ds(s * TROW, TROW), :],
                    aggh.at[c, pl.ds(s * TROW, TROW), :])

    @pl.when(s == SC_TILES - 1)
    def _():
        pltpu.sync_copy(
            agg_sh.at[pl.ds(SC_TILES * TROW, NN - SC_TILES * TROW), :],
            aggh.at[c, pl.ds(SC_TILES * TROW, NN - SC_TILES * TROW), :])


def _embed_body(nf, w, b, ns, h_out, x_out):
    h = jnp.dot(nf[...], w[...], preferred_element_type=jnp.float32) + b[...]
    h_out[...] = h
    xs = (h * ns[...]).astype(jnp.bfloat16)
    x_out[0] = xs[:, :HALF]
    x_out[1] = xs[:, HALF:]


def _embed_call(nf, w, b, ns):
    return pl.pallas_call(
        _embed_body,
        out_shape=[
            jax.ShapeDtypeStruct((NN, HH), jnp.float32),
            jax.ShapeDtypeStruct((SC_CORES, NN, HALF), jnp.bfloat16),
        ],
    )(nf, w, b, ns)


def _sigma_perm():
    m = lax.broadcasted_iota(jnp.int32, (1, HH), 1)
    blk = m >> 5
    t = m & 31
    src = 32 * blk + jnp.where(t < 16, 2 * t, 2 * (t - 16) + 1)
    r = lax.broadcasted_iota(jnp.int32, (HH, 1), 0)
    return (r == src).astype(jnp.float32)


def _layer_core(agg, h_in, nd, nns, w, b, gamma, beta):
    a = agg[...]
    aggf = jnp.concatenate([a[0], a[1]], axis=1) * nd[...]
    dnp = (((0,), (0,)), ((), ()))
    w_eff = lax.dot_general(_sigma_perm(), w[...], dnp,
                            preferred_element_type=jnp.float32)
    hc = jnp.dot(aggf, w_eff, preferred_element_type=jnp.float32) + b[...]
    hc = hc * nns[...]
    mean = jnp.mean(hc, axis=0, keepdims=True)
    cent = hc - mean
    var = jnp.mean(cent * cent, axis=0, keepdims=True)
    hn = cent * lax.rsqrt(var + 1e-5) * gamma[...] + beta[...]
    return h_in[...] + jnp.maximum(hn, 0.0)


def _layer_body(agg, h_in, nd, nns, w, b, gamma, beta, ns, h_out, x_out):
    h = _layer_core(agg, h_in, nd, nns, w, b, gamma, beta)
    h_out[...] = h
    xs = (h * ns[...]).astype(jnp.bfloat16)
    x_out[0] = xs[:, :HALF]
    x_out[1] = xs[:, HALF:]


def _layer_call(agg, h_in, nd, nns, w, b, gamma, beta, ns):
    return pl.pallas_call(
        _layer_body,
        out_shape=[
            jax.ShapeDtypeStruct((NN, HH), jnp.float32),
            jax.ShapeDtypeStruct((SC_CORES, NN, HALF), jnp.bfloat16),
        ],
    )(agg, h_in, nd, nns, w, b, gamma, beta, ns)


def _last_body(agg, h_in, nd, nns, w, b, gamma, beta,
               gid, w0, b0, w1, b1, w2, b2, out):
    h = _layer_core(agg, h_in, nd, nns, w, b, gamma, beta)
    iota = lax.broadcasted_iota(jnp.int32, (1, GG), 1)
    onehot = (gid[...] == iota).astype(jnp.float32)
    dn = (((0,), (0,)), ((), ()))
    hsum = lax.dot_general(onehot, h, dn,
                           preferred_element_type=jnp.float32)
    counts = lax.dot_general(onehot, jnp.ones((NN, 1), jnp.float32), dn,
                             preferred_element_type=jnp.float32)
    hg = hsum / jnp.maximum(counts, 1.0)
    y = jnp.maximum(jnp.dot(hg, w0[...], preferred_element_type=jnp.float32)
                    + b0[...], 0.0)
    y = jnp.maximum(jnp.dot(y, w1[...], preferred_element_type=jnp.float32)
                    + b1[...], 0.0)
    out[...] = jnp.dot(y, w2[...], preferred_element_type=jnp.float32) + b2[...]


def _last_call(agg, h_in, nd, nns, w, b, gamma, beta,
               gid, w0, b0, w1, b1, w2, b2):
    return pl.pallas_call(
        _last_body,
        out_shape=jax.ShapeDtypeStruct((GG, NCLS), jnp.float32),
    )(agg, h_in, nd, nns, w, b, gamma, beta, gid, w0, b0, w1, b1, w2, b2)


def _readout_body(h, gid, w0, b0, w1, b1, w2, b2, out):
    iota = lax.broadcasted_iota(jnp.int32, (1, GG), 1)
    onehot = (gid[...] == iota).astype(jnp.float32)
    dn = (((0,), (0,)), ((), ()))
    hsum = lax.dot_general(onehot, h[...], dn,
                           preferred_element_type=jnp.float32)
    counts = lax.dot_general(onehot, jnp.ones((NN, 1), jnp.float32), dn,
                             preferred_element_type=jnp.float32)
    hg = hsum / jnp.maximum(counts, 1.0)
    y = jnp.maximum(jnp.dot(hg, w0[...], preferred_element_type=jnp.float32)
                    + b0[...], 0.0)
    y = jnp.maximum(jnp.dot(y, w1[...], preferred_element_type=jnp.float32)
                    + b1[...], 0.0)
    out[...] = jnp.dot(y, w2[...], preferred_element_type=jnp.float32) + b2[...]


def _readout_call(h, gid, w0, b0, w1, b1, w2, b2):
    return pl.pallas_call(
        _readout_body,
        out_shape=jax.ShapeDtypeStruct((GG, NCLS), jnp.float32),
    )(h, gid, w0, b0, w1, b1, w2, b2)


def kernel(nodes_feat, nodes_num_norm_sqrt, edges_feat, edges_num_norm_sqrt,
           W_embed, b_embed, Ws, bs, gammas, betas,
           W_r0, b_r0, W_r1, b_r1, W_r2, b_r2,
           edge_index, graph_ids):
    pad = jnp.full((2, EPAD - EE), NN, dtype=jnp.int32)
    ei = jnp.concatenate([edge_index, pad], axis=1)
    srcm = ei[0].reshape(NCH, CHUNK)
    dstm = ei[1].reshape(NCH, CHUNK)

    deg = _deg_kernel(srcm, dstm)
    deg_out = deg[0, 0, :NN, 0] + deg[1, 0, :NN, 0]
    deg_in = deg[0, 1, :NN, 0] + deg[1, 1, :NN, 0]
    norm_src = lax.rsqrt(jnp.maximum(deg_out, 1.0)).reshape(NN, 1)
    norm_dst = lax.rsqrt(jnp.maximum(deg_in, 1.0)).reshape(NN, 1)

    h, x = _embed_call(nodes_feat, W_embed, b_embed.reshape(1, HH), norm_src)
    for i in range(LL - 1):
        agg = _mp_kernel(x, srcm, dstm)
        h, x = _layer_call(agg, h, norm_dst, nodes_num_norm_sqrt,
                           Ws[i], bs[i].reshape(1, HH),
                           gammas[i].reshape(1, HH), betas[i].reshape(1, HH),
                           norm_src)

    agg = _mp_kernel(x, srcm, dstm)
    i = LL - 1
    return _last_call(agg, h, norm_dst, nodes_num_norm_sqrt,
                      Ws[i], bs[i].reshape(1, HH),
                      gammas[i].reshape(1, HH), betas[i].reshape(1, HH),
                      graph_ids.reshape(NN, 1),
                      W_r0, b_r0.reshape(1, -1),
                      W_r1, b_r1.reshape(1, -1),
                      W_r2, b_r2.reshape(1, -1))
